# baseline (speedup 1.0000x reference)
import jax
import jax.numpy as jnp
from jax import lax
from jax.experimental import pallas as pl
from jax.experimental.pallas import tpu as pltpu
from jax.experimental.pallas import tpu_sc as plsc

DIM = 16


def _mf_body(bpw, nc, lanes,
             u_idx, i_idx, u_tab, i_tab, out,
             uidx_v, iidx_v, urows_v, irows_v, dots_v, sem_u, sem_i):
    wid = lax.axis_index("s") * nc + lax.axis_index("c")
    base = wid * bpw
    pltpu.sync_copy(u_idx.at[pl.ds(base, bpw)], uidx_v)
    pltpu.sync_copy(i_idx.at[pl.ds(base, bpw)], iidx_v)
    cu = pltpu.async_copy(u_tab.at[uidx_v], urows_v, sem_u)
    ci = pltpu.async_copy(i_tab.at[iidx_v], irows_v, sem_i)
    cu.wait()
    ci.wait()

    lane = lax.iota(jnp.int32, lanes)

    def chunk(c, carry):
        acc = jnp.zeros((lanes,), jnp.float32)
        for k in range(lanes):
            j = c * lanes + k
            s = jnp.sum(urows_v[j] * irows_v[j])
            acc = jnp.where(lane == k, s, acc)
        y = 1.0 / (1.0 + jnp.exp(-acc))
        dots_v[pl.ds(c * lanes, lanes)] = y
        return carry

    lax.fori_loop(0, bpw // lanes, chunk, 0)
    pltpu.sync_copy(dots_v, out.at[pl.ds(base, bpw)])


def kernel(userIdx, itemIdx, uEmbed, iEmbed):
    batch = userIdx.shape[0]
    info = plsc.get_sparse_core_info()
    nc, ns, lanes = info.num_cores, info.num_subcores, info.num_lanes
    nw = nc * ns
    bpw = batch // nw

    mesh = plsc.VectorSubcoreMesh(core_axis_name="c", subcore_axis_name="s")
    f = pl.kernel(
        lambda *refs: _mf_body(bpw, nc, lanes, *refs),
        mesh=mesh,
        compiler_params=pltpu.CompilerParams(
            needs_layout_passes=False, use_tc_tiling_on_sc=False),
        out_type=jax.ShapeDtypeStruct((batch,), jnp.float32),
        scratch_types=[
            pltpu.VMEM((bpw,), jnp.int32),
            pltpu.VMEM((bpw,), jnp.int32),
            pltpu.VMEM((bpw, DIM), jnp.float32),
            pltpu.VMEM((bpw, DIM), jnp.float32),
            pltpu.VMEM((bpw,), jnp.float32),
            pltpu.SemaphoreType.DMA,
            pltpu.SemaphoreType.DMA,
        ],
    )
    return f(userIdx.astype(jnp.int32), itemIdx.astype(jnp.int32),
             uEmbed, iEmbed)
